# Initial kernel scaffold; baseline (speedup 1.0000x reference)
import jax
import jax.numpy as jnp
from jax.experimental import pallas as pl


def kernel(x, edge_index, eps1, W1_1, b1_1, W1_2, b1_2, eps2, W2_1, b2_1, W2_2, b2_2, eps3, W3_1, b3_1, W3_2, b3_2):
    raise NotImplementedError("write your pallas kernel here")


# speedup vs baseline: 2.8701x; 2.8701x over previous
import functools

import jax
import jax.numpy as jnp
from jax import lax
from jax.experimental import pallas as pl
from jax.experimental.pallas import tpu as pltpu
from jax.experimental.pallas import tpu_sc as plsc

N = 10000
E = 320000
D = 128

NC = 2
NS = 16
NW = NC * NS

CHUNK = 128
EPW = 10240
E_PAD = EPW * NW
CPT = EPW // CHUNK
N_PAD = 10240
ZROWS = N_PAD // NS


def _sc_agg_body(x_hbm, src_hbm, dst_hbm, out_hbm, src_v, dst_v, buf, acc, sem):
    c = lax.axis_index("c")
    s = lax.axis_index("s")
    wid = s * NC + c

    pltpu.sync_copy(src_hbm.at[pl.ds(wid * CPT, CPT)], src_v)
    pltpu.sync_copy(dst_hbm.at[pl.ds(wid * CPT, CPT)], dst_v)

    def zrow(i, _):
        def zcol(j, _):
            buf[0, i, pl.ds(j * 16, 16)] = jnp.zeros((16,), jnp.float32)
            return 0
        return lax.fori_loop(0, D // 16, zcol, 0)
    lax.fori_loop(0, CHUNK, zrow, 0)
    for k in range(ZROWS // CHUNK):
        pltpu.sync_copy(buf.at[0], acc.at[pl.ds(s * ZROWS + k * CHUNK, CHUNK)])
    plsc.subcore_barrier()

    def body(j, _):
        pltpu.async_copy(x_hbm.at[src_v.at[j]], buf.at[0], sem).wait()
        pltpu.sync_copy(buf.at[0], acc.at[dst_v.at[j]], add=True)
        return 0
    lax.fori_loop(0, CPT, body, 0)

    plsc.subcore_barrier()
    @pl.when(s < NS - 1)
    def _():
        pltpu.sync_copy(acc.at[pl.ds(s * ZROWS, ZROWS)],
                        out_hbm.at[c, pl.ds(s * ZROWS, ZROWS)])

    @pl.when(s == NS - 1)
    def _():
        pltpu.sync_copy(acc.at[pl.ds((NS - 1) * ZROWS, N - (NS - 1) * ZROWS)],
                        out_hbm.at[c, pl.ds((NS - 1) * ZROWS,
                                            N - (NS - 1) * ZROWS)])


@functools.partial(
    pl.kernel,
    out_type=jax.ShapeDtypeStruct((NC, N, D), jnp.float32),
    mesh=plsc.VectorSubcoreMesh(core_axis_name="c", subcore_axis_name="s"),
    scratch_types=[
        pltpu.VMEM((CPT, CHUNK), jnp.int32),
        pltpu.VMEM((CPT, CHUNK), jnp.int32),
        pltpu.VMEM((1, CHUNK, D), jnp.float32),
        pltpu.VMEM_SHARED((N_PAD, D), jnp.float32),
        pltpu.SemaphoreType.DMA,
    ],
)
def _sc_agg(x_hbm, src_hbm, dst_hbm, out_hbm, src_v, dst_v, buf, acc, sem):
    _sc_agg_body(x_hbm, src_hbm, dst_hbm, out_hbm, src_v, dst_v, buf, acc, sem)


BR = 2000


def _mlp_body(relu_out, h_ref, p_ref, eps_ref, wa_ref, ba_ref, wb_ref, bb_ref,
              o_ref):
    t = h_ref[...] * (1.0 + eps_ref[0, 0]) + p_ref[0] + p_ref[1]
    t = jnp.dot(t, wa_ref[...], preferred_element_type=jnp.float32,
                precision=lax.Precision.HIGHEST)
    t = jnp.maximum(t + ba_ref[...], 0.0)
    t = jnp.dot(t, wb_ref[...], preferred_element_type=jnp.float32,
                precision=lax.Precision.HIGHEST)
    t = t + bb_ref[...]
    if relu_out:
        t = jnp.maximum(t, 0.0)
    o_ref[...] = t


def _tc_mlp(h, parts, eps, Wa, ba, Wb, bb, relu_out):
    return pl.pallas_call(
        functools.partial(_mlp_body, relu_out),
        grid=(N // BR,),
        in_specs=[
            pl.BlockSpec((BR, D), lambda i: (i, 0)),
            pl.BlockSpec((NC, BR, D), lambda i: (0, i, 0)),
            pl.BlockSpec((1, 1), lambda i: (0, 0)),
            pl.BlockSpec((D, D), lambda i: (0, 0)),
            pl.BlockSpec((1, D), lambda i: (0, 0)),
            pl.BlockSpec((D, D), lambda i: (0, 0)),
            pl.BlockSpec((1, D), lambda i: (0, 0)),
        ],
        out_specs=pl.BlockSpec((BR, D), lambda i: (i, 0)),
        out_shape=jax.ShapeDtypeStruct((N, D), jnp.float32),
    )(h, parts, eps, Wa, ba, Wb, bb)


def kernel(x, edge_index, eps1, W1_1, b1_1, W1_2, b1_2, eps2, W2_1, b2_1,
           W2_2, b2_2, eps3, W3_1, b3_1, W3_2, b3_2):
    src = edge_index[0]
    dst = edge_index[1]
    pad = E_PAD - E
    src2d = jnp.concatenate(
        [src, jnp.zeros((pad,), jnp.int32)]).reshape(E_PAD // CHUNK, CHUNK)
    dst2d = jnp.concatenate(
        [dst, jnp.full((pad,), N, jnp.int32)]).reshape(E_PAD // CHUNK, CHUNK)

    layers = [
        (eps1, W1_1, b1_1, W1_2, b1_2, True),
        (eps2, W2_1, b2_1, W2_2, b2_2, True),
        (eps3, W3_1, b3_1, W3_2, b3_2, False),
    ]
    h = x
    for eps, Wa, ba, Wb, bb, relu_out in layers:
        parts = _sc_agg(h, src2d, dst2d)
        h = _tc_mlp(h, parts, eps.reshape(1, 1), Wa, ba.reshape(1, D), Wb,
                    bb.reshape(1, D), relu_out)
    return h


# speedup vs baseline: 3.2569x; 1.1348x over previous
import functools

import jax
import jax.numpy as jnp
from jax import lax
from jax.experimental import pallas as pl
from jax.experimental.pallas import tpu as pltpu
from jax.experimental.pallas import tpu_sc as plsc

N = 10000
E = 320000
D = 128

NC = 2
NS = 16
NW = NC * NS

CHUNK = 128
EPW = 10240
E_PAD = EPW * NW
CPT = EPW // CHUNK
N_PAD = 10240
ZROWS = N_PAD // NS
G = 8
NGRP = CPT // G
NBUF = 2


def _sc_agg_body(x_hbm, src_hbm, dst_hbm, out_hbm, src_v, dst_v, buf, acc,
                 gsem0, gsem1, isem_s, isem_d):
    c = lax.axis_index("c")
    s = lax.axis_index("s")
    wid = s * NC + c
    base = wid * CPT
    gsems = (gsem0, gsem1)

    def zrow(i, _):
        def zcol(j, _):
            buf[0, i, pl.ds(j * 16, 16)] = jnp.zeros((16,), jnp.float32)
            return 0
        return lax.fori_loop(0, D // 16, zcol, 0)
    lax.fori_loop(0, CHUNK, zrow, 0)
    for k in range(ZROWS // CHUNK):
        pltpu.sync_copy(buf.at[0], acc.at[pl.ds(s * ZROWS + k * CHUNK, CHUNK)])

    pltpu.sync_copy(src_hbm.at[pl.ds(base, G)], src_v.at[0])
    pltpu.sync_copy(dst_hbm.at[pl.ds(base, G)], dst_v.at[0])
    plsc.subcore_barrier()
    for b in range(NBUF):
        pltpu.async_copy(x_hbm.at[src_v.at[0, b]], buf.at[b], gsems[b])

    for g in range(NGRP):
        sl = g % 2
        nsl = 1 - sl
        if g + 1 < NGRP:
            pltpu.async_copy(src_hbm.at[pl.ds(base + (g + 1) * G, G)],
                             src_v.at[nsl], isem_s)
            pltpu.async_copy(dst_hbm.at[pl.ds(base + (g + 1) * G, G)],
                             dst_v.at[nsl], isem_d)
        if g > 0:
            pltpu.make_async_copy(dst_hbm.at[pl.ds(base + g * G, G)],
                                  dst_v.at[sl], isem_d).wait()
        for k in range(G):
            j = g * G + k
            b = j % NBUF
            pltpu.make_async_copy(x_hbm.at[src_v.at[sl, k]], buf.at[b],
                                  gsems[b]).wait()
            pltpu.sync_copy(buf.at[b], acc.at[dst_v.at[sl, k]], add=True)
            jn = j + NBUF
            if jn < CPT:
                gn, kn = divmod(jn, G)
                if gn != g and kn == 0:
                    pltpu.make_async_copy(
                        src_hbm.at[pl.ds(base + gn * G, G)],
                        src_v.at[gn % 2], isem_s).wait()
                pltpu.async_copy(x_hbm.at[src_v.at[gn % 2, kn]], buf.at[b],
                                 gsems[b])

    plsc.subcore_barrier()
    @pl.when(s < NS - 1)
    def _():
        pltpu.sync_copy(acc.at[pl.ds(s * ZROWS, ZROWS)],
                        out_hbm.at[c, pl.ds(s * ZROWS, ZROWS)])

    @pl.when(s == NS - 1)
    def _():
        pltpu.sync_copy(acc.at[pl.ds((NS - 1) * ZROWS, N - (NS - 1) * ZROWS)],
                        out_hbm.at[c, pl.ds((NS - 1) * ZROWS,
                                            N - (NS - 1) * ZROWS)])


@functools.partial(
    pl.kernel,
    out_type=jax.ShapeDtypeStruct((NC, N, D), jnp.float32),
    mesh=plsc.VectorSubcoreMesh(core_axis_name="c", subcore_axis_name="s"),
    scratch_types=[
        pltpu.VMEM((2, G, CHUNK), jnp.int32),
        pltpu.VMEM((2, G, CHUNK), jnp.int32),
        pltpu.VMEM((NBUF, CHUNK, D), jnp.float32),
        pltpu.VMEM_SHARED((N_PAD, D), jnp.float32),
        pltpu.SemaphoreType.DMA,
        pltpu.SemaphoreType.DMA,
        pltpu.SemaphoreType.DMA,
        pltpu.SemaphoreType.DMA,
    ],
)
def _sc_agg(x_hbm, src_hbm, dst_hbm, out_hbm, src_v, dst_v, buf, acc,
            gsem0, gsem1, isem_s, isem_d):
    _sc_agg_body(x_hbm, src_hbm, dst_hbm, out_hbm, src_v, dst_v, buf, acc,
                 gsem0, gsem1, isem_s, isem_d)


BR = 2000


def _mlp_body(relu_out, h_ref, p_ref, eps_ref, wa_ref, ba_ref, wb_ref, bb_ref,
              o_ref):
    t = h_ref[...] * (1.0 + eps_ref[0, 0]) + p_ref[0] + p_ref[1]
    t = jnp.dot(t, wa_ref[...], preferred_element_type=jnp.float32,
                precision=lax.Precision.HIGHEST)
    t = jnp.maximum(t + ba_ref[...], 0.0)
    t = jnp.dot(t, wb_ref[...], preferred_element_type=jnp.float32,
                precision=lax.Precision.HIGHEST)
    t = t + bb_ref[...]
    if relu_out:
        t = jnp.maximum(t, 0.0)
    o_ref[...] = t


def _tc_mlp(h, parts, eps, Wa, ba, Wb, bb, relu_out):
    return pl.pallas_call(
        functools.partial(_mlp_body, relu_out),
        grid=(N // BR,),
        in_specs=[
            pl.BlockSpec((BR, D), lambda i: (i, 0)),
            pl.BlockSpec((NC, BR, D), lambda i: (0, i, 0)),
            pl.BlockSpec((1, 1), lambda i: (0, 0)),
            pl.BlockSpec((D, D), lambda i: (0, 0)),
            pl.BlockSpec((1, D), lambda i: (0, 0)),
            pl.BlockSpec((D, D), lambda i: (0, 0)),
            pl.BlockSpec((1, D), lambda i: (0, 0)),
        ],
        out_specs=pl.BlockSpec((BR, D), lambda i: (i, 0)),
        out_shape=jax.ShapeDtypeStruct((N, D), jnp.float32),
    )(h, parts, eps, Wa, ba, Wb, bb)


def kernel(x, edge_index, eps1, W1_1, b1_1, W1_2, b1_2, eps2, W2_1, b2_1,
           W2_2, b2_2, eps3, W3_1, b3_1, W3_2, b3_2):
    src = edge_index[0]
    dst = edge_index[1]
    pad = E_PAD - E
    src2d = jnp.concatenate(
        [src, jnp.zeros((pad,), jnp.int32)]).reshape(E_PAD // CHUNK, CHUNK)
    dst2d = jnp.concatenate(
        [dst, jnp.full((pad,), N, jnp.int32)]).reshape(E_PAD // CHUNK, CHUNK)

    layers = [
        (eps1, W1_1, b1_1, W1_2, b1_2, True),
        (eps2, W2_1, b2_1, W2_2, b2_2, True),
        (eps3, W3_1, b3_1, W3_2, b3_2, False),
    ]
    h = x
    for eps, Wa, ba, Wb, bb, relu_out in layers:
        parts = _sc_agg(h, src2d, dst2d)
        h = _tc_mlp(h, parts, eps.reshape(1, 1), Wa, ba.reshape(1, D), Wb,
                    bb.reshape(1, D), relu_out)
    return h
